# baseline (speedup 1.0000x reference)
import functools

import jax
import jax.numpy as jnp
from jax import lax
from jax.experimental import pallas as pl
from jax.experimental.pallas import tpu as pltpu
from jax.experimental.pallas import tpu_sc as plsc

_N_BINS = 512
_N_DIFFS = 16384
_NC, _NS, _L = 2, 16, 16
_NW = _NC * _NS
_ROWS = _N_DIFFS // _NW
_STEPS = _ROWS // _L
_TSTEPS = _N_BINS // _L

_LN2 = 0.6931471805599453


def _ln(r):
    bits = plsc.bitcast(r, jnp.int32)
    e = (bits >> 23) - 127
    m = plsc.bitcast((bits & 0x007FFFFF) | 0x3F800000, jnp.float32)
    z = (m - 1.0) / (m + 1.0)
    z2 = z * z
    p = 1.0 / 9.0
    p = p * z2 + 1.0 / 7.0
    p = p * z2 + 1.0 / 5.0
    p = p * z2 + 1.0 / 3.0
    p = p * z2 + 1.0
    return 2.0 * z * p + e.astype(jnp.float32) * _LN2


@functools.partial(
    pl.kernel,
    out_type=jax.ShapeDtypeStruct((_N_DIFFS,), jnp.float32),
    mesh=plsc.VectorSubcoreMesh(core_axis_name="c", subcore_axis_name="s"),
    compiler_params=pltpu.CompilerParams(needs_layout_passes=False),
    scratch_types=[
        pltpu.VMEM((_N_BINS,), jnp.float32),
        pltpu.VMEM((112,), jnp.float32),
        pltpu.VMEM((3 * _N_BINS,), jnp.float32),
        pltpu.VMEM((3 * _ROWS,), jnp.int32),
        pltpu.VMEM((_ROWS,), jnp.float32),
        pltpu.SemaphoreType.DMA,
        pltpu.SemaphoreType.DMA,
        pltpu.SemaphoreType.DMA,
        pltpu.SemaphoreType.DMA,
        pltpu.SemaphoreType.DMA,
        pltpu.SemaphoreType.DMA,
        pltpu.SemaphoreType.DMA,
        pltpu.SemaphoreType.DMA,
    ],
)
def _sc_model(bc_hbm, l1, u1, l2, u2, l3, u3, idx_hbm, out_hbm,
              bc_v, scal_v, tab_v, idx_v, out_v, sem_b, sem_i,
              sem_p0, sem_p1, sem_p2, sem_p3, sem_p4, sem_p5):
    wid = lax.axis_index("s") * _NC + lax.axis_index("c")
    base = wid * _ROWS
    cp_i = pltpu.async_copy(idx_hbm.at[pl.ds(3 * base, 3 * _ROWS)], idx_v, sem_i)
    cp_b = pltpu.async_copy(bc_hbm, bc_v, sem_b)
    sems = (sem_p0, sem_p1, sem_p2, sem_p3, sem_p4, sem_p5)
    cps = [
        pltpu.async_copy(p_hbm, scal_v.at[pl.ds(16 * (k + 1), 1)], sems[k])
        for k, p_hbm in enumerate((l1, u1, l2, u2, l3, u3))
    ]
    cp_b.wait()
    for cp in cps:
        cp.wait()

    ubs, invs = [], []
    for j in range(3):
        vl = plsc.load_gather(scal_v, [jnp.full((_L,), 32 * j + 16, jnp.int32)])
        vu = plsc.load_gather(scal_v, [jnp.full((_L,), 32 * j + 32, jnp.int32)])
        lo = jnp.minimum(vl, vu)
        hi = jnp.maximum(vl, vu)
        ubs.append(hi)
        invs.append(1.0 / (hi - lo + 0.0001))

    @plsc.parallel_loop(0, _TSTEPS, step=1, unroll=4)
    def tbody(k):
        t = bc_v[pl.ds(_L * k, _L)]
        logit = _ln(t / (1.0 - t))
        for j in range(3):
            x = (ubs[j] - logit) * invs[j]
            tab_v[pl.ds(j * _N_BINS + _L * k, _L)] = 1.0 / (1.0 + jnp.exp(-x))

    cp_i.wait()
    lanes3 = lax.iota(jnp.int32, _L) * 3
    half = _STEPS // 2

    def gbody(i):
        p = None
        for j in range(3):
            bidx = plsc.load_gather(idx_v, [lanes3 + (i * (3 * _L) + j)])
            pj = plsc.load_gather(tab_v, [bidx + (j * _N_BINS)])
            p = pj if p is None else p * pj
        out_v[pl.ds(i * _L, _L)] = p

    plsc.parallel_loop(0, half, step=1, unroll=4)(gbody)
    cp_o1 = pltpu.async_copy(out_v.at[pl.ds(0, half * _L)],
                             out_hbm.at[pl.ds(base, half * _L)], sem_b)
    plsc.parallel_loop(half, _STEPS, step=1, unroll=4)(gbody)
    cp_o2 = pltpu.async_copy(out_v.at[pl.ds(half * _L, half * _L)],
                             out_hbm.at[pl.ds(base + half * _L, half * _L)], sem_i)
    cp_o1.wait()
    cp_o2.wait()


def kernel(bin_centers, observation_probability_index, operator_number,
           lower_bound_1, upper_bound_1, lower_bound_2, upper_bound_2,
           lower_bound_3, upper_bound_3):
    del operator_number
    return _sc_model(bin_centers, lower_bound_1, upper_bound_1,
                     lower_bound_2, upper_bound_2, lower_bound_3,
                     upper_bound_3, observation_probability_index.reshape(-1))


# speedup vs baseline: 1.0261x; 1.0261x over previous
import functools

import jax
import jax.numpy as jnp
from jax import lax
from jax.experimental import pallas as pl
from jax.experimental.pallas import tpu as pltpu
from jax.experimental.pallas import tpu_sc as plsc

_N_BINS = 512
_N_DIFFS = 16384
_NC, _NS, _L = 2, 16, 16
_NW = _NC * _NS
_ROWS = _N_DIFFS // _NW
_STEPS = _ROWS // _L
_TSTEPS = _N_BINS // _L

_LN2 = 0.6931471805599453


def _ln(r):
    bits = plsc.bitcast(r, jnp.int32)
    e = (bits >> 23) - 127
    m = plsc.bitcast((bits & 0x007FFFFF) | 0x3F800000, jnp.float32)
    z = (m - 1.0) / (m + 1.0)
    z2 = z * z
    p = 1.0 / 9.0
    p = p * z2 + 1.0 / 7.0
    p = p * z2 + 1.0 / 5.0
    p = p * z2 + 1.0 / 3.0
    p = p * z2 + 1.0
    return 2.0 * z * p + e.astype(jnp.float32) * _LN2


@functools.partial(
    pl.kernel,
    out_type=jax.ShapeDtypeStruct((_N_DIFFS,), jnp.float32),
    mesh=plsc.VectorSubcoreMesh(core_axis_name="c", subcore_axis_name="s"),
    compiler_params=pltpu.CompilerParams(needs_layout_passes=False),
    scratch_types=[
        pltpu.VMEM((_N_BINS,), jnp.float32),
        pltpu.VMEM((112,), jnp.float32),
        pltpu.VMEM((3 * _N_BINS,), jnp.float32),
        pltpu.VMEM((3 * _ROWS,), jnp.int32),
        pltpu.VMEM((_ROWS,), jnp.float32),
        pltpu.VMEM_SHARED((3 * _N_BINS,), jnp.float32),
        pltpu.SemaphoreType.DMA,
        pltpu.SemaphoreType.DMA,
        pltpu.SemaphoreType.DMA,
        pltpu.SemaphoreType.DMA,
        pltpu.SemaphoreType.DMA,
        pltpu.SemaphoreType.DMA,
        pltpu.SemaphoreType.DMA,
        pltpu.SemaphoreType.DMA,
    ],
)
def _sc_model(bc_hbm, l1, u1, l2, u2, l3, u3, idx_hbm, out_hbm,
              bc_v, scal_v, tab_v, idx_v, out_v, tab_sh, sem_b, sem_i,
              sem_p0, sem_p1, sem_p2, sem_p3, sem_p4, sem_p5):
    sid = lax.axis_index("s")
    wid = sid * _NC + lax.axis_index("c")
    base = wid * _ROWS
    cp_i = pltpu.async_copy(idx_hbm.at[pl.ds(3 * base, 3 * _ROWS)], idx_v, sem_i)

    @pl.when(sid == 0)
    def _build():
        cp_b = pltpu.async_copy(bc_hbm, bc_v, sem_b)
        sems = (sem_p0, sem_p1, sem_p2, sem_p3, sem_p4, sem_p5)
        cps = [
            pltpu.async_copy(p_hbm, scal_v.at[pl.ds(16 * (k + 1), 1)], sems[k])
            for k, p_hbm in enumerate((l1, u1, l2, u2, l3, u3))
        ]
        cp_b.wait()
        for cp in cps:
            cp.wait()

        ubs, invs = [], []
        for j in range(3):
            vl = plsc.load_gather(scal_v, [jnp.full((_L,), 32 * j + 16, jnp.int32)])
            vu = plsc.load_gather(scal_v, [jnp.full((_L,), 32 * j + 32, jnp.int32)])
            lo = jnp.minimum(vl, vu)
            hi = jnp.maximum(vl, vu)
            ubs.append(hi)
            invs.append(1.0 / (hi - lo + 0.0001))

        @plsc.parallel_loop(0, _TSTEPS, step=1, unroll=4)
        def tbody(k):
            t = bc_v[pl.ds(_L * k, _L)]
            logit = _ln(t / (1.0 - t))
            for j in range(3):
                x = (ubs[j] - logit) * invs[j]
                tab_v[pl.ds(j * _N_BINS + _L * k, _L)] = 1.0 / (1.0 + jnp.exp(-x))

        pltpu.sync_copy(tab_v, tab_sh)

    plsc.subcore_barrier()

    @pl.when(sid != 0)
    def _fetch():
        pltpu.sync_copy(tab_sh, tab_v)

    cp_i.wait()
    lanes3 = lax.iota(jnp.int32, _L) * 3
    half = _STEPS // 2

    def gbody(i):
        p = None
        for j in range(3):
            bidx = plsc.load_gather(idx_v, [lanes3 + (i * (3 * _L) + j)])
            pj = plsc.load_gather(tab_v, [bidx + (j * _N_BINS)])
            p = pj if p is None else p * pj
        out_v[pl.ds(i * _L, _L)] = p

    plsc.parallel_loop(0, half, step=1, unroll=4)(gbody)
    cp_o1 = pltpu.async_copy(out_v.at[pl.ds(0, half * _L)],
                             out_hbm.at[pl.ds(base, half * _L)], sem_b)
    plsc.parallel_loop(half, _STEPS, step=1, unroll=4)(gbody)
    cp_o2 = pltpu.async_copy(out_v.at[pl.ds(half * _L, half * _L)],
                             out_hbm.at[pl.ds(base + half * _L, half * _L)], sem_i)
    cp_o1.wait()
    cp_o2.wait()


def kernel(bin_centers, observation_probability_index, operator_number,
           lower_bound_1, upper_bound_1, lower_bound_2, upper_bound_2,
           lower_bound_3, upper_bound_3):
    del operator_number
    return _sc_model(bin_centers, lower_bound_1, upper_bound_1,
                     lower_bound_2, upper_bound_2, lower_bound_3,
                     upper_bound_3, observation_probability_index.reshape(-1))


# speedup vs baseline: 1.0295x; 1.0033x over previous
import functools

import jax
import jax.numpy as jnp
from jax import lax
from jax.experimental import pallas as pl
from jax.experimental.pallas import tpu as pltpu
from jax.experimental.pallas import tpu_sc as plsc

_N_BINS = 512
_N_DIFFS = 16384
_NC, _NS, _L = 2, 16, 16
_NW = _NC * _NS
_ROWS = _N_DIFFS // _NW
_STEPS = _ROWS // _L
_TSTEPS = _N_BINS // _L

_LN2 = 0.6931471805599453


def _ln(r):
    bits = plsc.bitcast(r, jnp.int32)
    e = (bits >> 23) - 127
    m = plsc.bitcast((bits & 0x007FFFFF) | 0x3F800000, jnp.float32)
    z = (m - 1.0) / (m + 1.0)
    z2 = z * z
    p = 1.0 / 9.0
    p = p * z2 + 1.0 / 7.0
    p = p * z2 + 1.0 / 5.0
    p = p * z2 + 1.0 / 3.0
    p = p * z2 + 1.0
    return 2.0 * z * p + e.astype(jnp.float32) * _LN2


@functools.partial(
    pl.kernel,
    out_type=jax.ShapeDtypeStruct((_N_DIFFS,), jnp.float32),
    mesh=plsc.VectorSubcoreMesh(core_axis_name="c", subcore_axis_name="s"),
    compiler_params=pltpu.CompilerParams(needs_layout_passes=False),
    scratch_types=[
        pltpu.VMEM((_N_BINS,), jnp.float32),
        pltpu.VMEM((112,), jnp.float32),
        pltpu.VMEM((3 * _N_BINS,), jnp.float32),
        pltpu.VMEM((3 * _ROWS,), jnp.int32),
        pltpu.VMEM((_ROWS,), jnp.float32),
        pltpu.VMEM_SHARED((3 * _N_BINS,), jnp.float32),
        pltpu.SemaphoreType.DMA,
        pltpu.SemaphoreType.DMA,
        pltpu.SemaphoreType.DMA,
        pltpu.SemaphoreType.DMA,
        pltpu.SemaphoreType.DMA,
        pltpu.SemaphoreType.DMA,
        pltpu.SemaphoreType.DMA,
        pltpu.SemaphoreType.DMA,
    ],
)
def _sc_model(bc_hbm, l1, u1, l2, u2, l3, u3, idx_hbm, out_hbm,
              bc_v, scal_v, tab_v, idx_v, out_v, tab_sh, sem_b, sem_i,
              sem_p0, sem_p1, sem_p2, sem_p3, sem_p4, sem_p5):
    sid = lax.axis_index("s")
    wid = sid * _NC + lax.axis_index("c")
    base = wid * _ROWS
    cp_i = pltpu.async_copy(idx_hbm.at[pl.ds(3 * base, 3 * _ROWS)], idx_v, sem_i)

    @pl.when(sid == 0)
    def _build():
        cp_b = pltpu.async_copy(bc_hbm, bc_v, sem_b)
        sems = (sem_p0, sem_p1, sem_p2, sem_p3, sem_p4, sem_p5)
        cps = [
            pltpu.async_copy(p_hbm, scal_v.at[pl.ds(16 * (k + 1), 1)], sems[k])
            for k, p_hbm in enumerate((l1, u1, l2, u2, l3, u3))
        ]
        cp_b.wait()
        for cp in cps:
            cp.wait()

        ubs, invs = [], []
        for j in range(3):
            vl = plsc.load_gather(scal_v, [jnp.full((_L,), 32 * j + 16, jnp.int32)])
            vu = plsc.load_gather(scal_v, [jnp.full((_L,), 32 * j + 32, jnp.int32)])
            lo = jnp.minimum(vl, vu)
            hi = jnp.maximum(vl, vu)
            ubs.append(hi)
            invs.append(1.0 / (hi - lo + 0.0001))

        @plsc.parallel_loop(0, _TSTEPS, step=1, unroll=4)
        def tbody(k):
            t = bc_v[pl.ds(_L * k, _L)]
            logit = _ln(t / (1.0 - t))
            for j in range(3):
                x = (ubs[j] - logit) * invs[j]
                tab_v[pl.ds(j * _N_BINS + _L * k, _L)] = 1.0 / (1.0 + jnp.exp(-x))

        pltpu.sync_copy(tab_v, tab_sh)

    plsc.subcore_barrier()

    @pl.when(sid != 0)
    def _fetch():
        pltpu.sync_copy(tab_sh, tab_v)

    cp_i.wait()
    lanes3 = lax.iota(jnp.int32, _L) * 3
    half = _STEPS // 2

    def gbody(i):
        p = None
        for j in range(3):
            bidx = plsc.load_gather(idx_v, [lanes3 + (i * (3 * _L) + j)])
            pj = plsc.load_gather(tab_v, [bidx + (j * _N_BINS)])
            p = pj if p is None else p * pj
        out_v[pl.ds(i * _L, _L)] = p

    plsc.parallel_loop(0, _STEPS, step=1, unroll=4)(gbody)
    pltpu.sync_copy(out_v, out_hbm.at[pl.ds(base, _ROWS)])


def kernel(bin_centers, observation_probability_index, operator_number,
           lower_bound_1, upper_bound_1, lower_bound_2, upper_bound_2,
           lower_bound_3, upper_bound_3):
    del operator_number
    return _sc_model(bin_centers, lower_bound_1, upper_bound_1,
                     lower_bound_2, upper_bound_2, lower_bound_3,
                     upper_bound_3, observation_probability_index.reshape(-1))
